# Initial kernel scaffold; baseline (speedup 1.0000x reference)
import jax
import jax.numpy as jnp
from jax.experimental import pallas as pl


def kernel(inputs):
    raise NotImplementedError("write your pallas kernel here")


# speedup vs baseline: 10.9800x; 10.9800x over previous
import jax
import jax.numpy as jnp
from jax.experimental import pallas as pl

BATCH = 16384
NUM_FIELDS = 100
EMBED_DIM = 16
BR = 128


def _body(x_ref, o_ref):
    x = x_ref[:]
    s2 = jnp.sum(x * x, axis=-1)
    w = jax.lax.rsqrt(1.0 + s2)
    S = jnp.sum(w, axis=-1)
    Q = jnp.sum(w * w, axis=-1)
    y = jnp.sum(x * w[:, :, None], axis=1)
    Y = jnp.sum(y * y, axis=-1)
    o_ref[:] = 5000.0 + 0.5 * (S * S - 2.0 * Q - Y) - 99.0 * S


def kernel(inputs):
    grid = (BATCH // BR,)
    return pl.pallas_call(
        _body,
        grid=grid,
        in_specs=[pl.BlockSpec((BR, NUM_FIELDS, EMBED_DIM), lambda i: (i, 0, 0))],
        out_specs=pl.BlockSpec((BR,), lambda i: (i,)),
        out_shape=jax.ShapeDtypeStruct((BATCH,), jnp.float32),
    )(inputs)
